# baseline (speedup 1.0000x reference)
import functools
import math

import jax
import jax.numpy as jnp
from jax import lax
from jax.experimental import pallas as pl
from jax.experimental.pallas import tpu as pltpu
from jax.experimental.pallas import tpu_sc as plsc

VOCAB = 1000000
DIM = 32
SEQ = 200
NB = 4096
HALF = 16

NC = 2
NS = 16
DPC = DIM // NC
BS = 128
SLICES = NB // (NS * BS)
BLK = 25
NBLK = SEQ // BLK
ROW_CHUNK = 62464
ROW_LAST = 999936 - (NS - 1) * ROW_CHUNK
TAIL = 128

_SCALE = math.sqrt(float(DIM))


def _pe_table(length, depth):
    half = depth // 2
    positions = jnp.arange(length, dtype=jnp.float32).reshape(-1, 1)
    depths = jnp.arange(half, dtype=jnp.float32).reshape(1, -1) / half
    angle = positions / (10000.0 ** depths)
    return jnp.concatenate([jnp.sin(angle), jnp.cos(angle)], axis=-1)


@functools.lru_cache(maxsize=None)
def _make_kernel():
    mesh = plsc.VectorSubcoreMesh(core_axis_name="c", subcore_axis_name="s")

    @functools.partial(
        pl.kernel,
        mesh=mesh,
        out_type=jax.ShapeDtypeStruct((SEQ, DIM, NB), jnp.float32),
        scratch_types=[
            pltpu.VMEM((SLICES, SEQ, BS), jnp.int32),
            pltpu.VMEM((BLK, BS), jnp.float32),
            pltpu.VMEM((BLK, BS), jnp.float32),
            pltpu.VMEM((SEQ // 8 + 1, 128), jnp.float32),
            pltpu.VMEM((TAIL,), jnp.float32),
            pltpu.VMEM((TAIL,), jnp.int32),
            pltpu.VMEM_SHARED((VOCAB,), jnp.float32),
            pltpu.SemaphoreType.DMA,
            pltpu.SemaphoreType.DMA,
            pltpu.SemaphoreType.DMA,
        ],
    )
    def emb(xt_hbm, tt_hbm, pe_hbm, out_hbm, idx_v, val_a, val_b,
            pe_v, tail_v, tidx_v, row_sh, gsem, osem_a, osem_b):
        cid = lax.axis_index("c")
        sid = lax.axis_index("s")
        scale = jnp.float32(_SCALE)
        vals = (val_a, val_b)
        osems = (osem_a, osem_b)

        for jj in range(TAIL // HALF):
            tidx_v[pl.ds(jj * HALF, HALF)] = (
                lax.iota(jnp.int32, HALF) + (VOCAB - TAIL + jj * HALF)
            )

        for h in range(SLICES):
            b0 = (sid * SLICES + h) * BS
            pltpu.sync_copy(xt_hbm.at[:, pl.ds(b0, BS)], idx_v.at[h])

        def unit(dd, carry):
            d = cid * DPC + dd

            plsc.subcore_barrier()
            off = sid * ROW_CHUNK

            @pl.when(sid < NS - 1)
            def _chunk():
                pltpu.sync_copy(
                    tt_hbm.at[d, pl.ds(off, ROW_CHUNK)],
                    row_sh.at[pl.ds(off, ROW_CHUNK)],
                )

            @pl.when(sid == NS - 1)
            def _last():
                pltpu.sync_copy(
                    tt_hbm.at[d, pl.ds((NS - 1) * ROW_CHUNK, ROW_LAST)],
                    row_sh.at[pl.ds((NS - 1) * ROW_CHUNK, ROW_LAST)],
                )

            pltpu.sync_copy(pe_hbm.at[d], pe_v)

            @pl.when(sid == NS - 1)
            def _tail():
                for jj in range(TAIL // HALF):
                    tail_v[pl.ds(jj * HALF, HALF)] = (
                        pe_v[SEQ // 8, pl.ds(jj * HALF, HALF)]
                    )
                pltpu.sync_copy(tail_v, row_sh.at[tidx_v])

            plsc.subcore_barrier()

            def hj(bi):
                return bi // NBLK, bi % NBLK

            def out_blk(bi):
                h, j = hj(bi)
                b0 = (sid * SLICES + h) * BS
                return out_hbm.at[pl.ds(j * BLK, BLK), d, pl.ds(b0, BS)]

            gathers = {}

            def fire(bi):
                h, j = hj(bi)
                dst = vals[bi % 2]
                cps = []
                for k in range(BLK):
                    cps.append(pltpu.async_copy(
                        row_sh.at[idx_v.at[h, j * BLK + k]], dst.at[k], gsem
                    ))
                gathers[bi] = cps

            def drain_out(bi):
                cp = pltpu.make_async_copy(out_blk(bi), vals[bi % 2],
                                           osems[bi % 2])
                if bi < 2:
                    @pl.when(dd > 0)
                    def _w():
                        cp.wait()
                else:
                    cp.wait()

            def compute(bi):
                _, j = hj(bi)
                dst = vals[bi % 2]

                def body(sl, c2):
                    s_glob = j * BLK + sl
                    pe_vec = pe_v[s_glob // 8,
                                  pl.ds((s_glob % 8) * HALF, HALF)]
                    for jj in range(BS // HALF):
                        dst[sl, pl.ds(jj * HALF, HALF)] = (
                            dst[sl, pl.ds(jj * HALF, HALF)] * scale + pe_vec
                        )
                    return c2

                lax.fori_loop(0, BLK, body, 0, unroll=False)

            def store(bi):
                pltpu.async_copy(vals[bi % 2], out_blk(bi), osems[bi % 2])

            nbi = SLICES * NBLK
            drain_out(0)
            fire(0)
            for bi in range(nbi):
                for cp in gathers[bi]:
                    cp.wait()
                if bi < nbi - 1:
                    drain_out(bi + 1)
                    fire(bi + 1)
                compute(bi)
                store(bi)
            return carry

        lax.fori_loop(0, DPC, unit, 0, unroll=False)
        pltpu.make_async_copy(
            out_hbm.at[pl.ds(0, BLK), 0, pl.ds(0, BS)], val_a, osem_a
        ).wait()
        pltpu.make_async_copy(
            out_hbm.at[pl.ds(0, BLK), 0, pl.ds(0, BS)], val_b, osem_b
        ).wait()

    return emb


def kernel(x, table):
    batch, seq = x.shape
    vocab, dim = table.shape
    assert (batch, seq, vocab, dim) == (NB, SEQ, VOCAB, DIM)
    pe = _pe_table(seq, dim)
    pe_b = jnp.broadcast_to(
        pe.T.reshape(dim, seq // 8, 8, 1), (dim, seq // 8, 8, HALF)
    ).reshape(dim, seq // 8, 8 * HALF)
    tail = table.T[:, VOCAB - TAIL:].reshape(dim, 1, TAIL)
    pe_b = jnp.concatenate([pe_b, tail], axis=1)
    out = _make_kernel()(x.T, table.T, pe_b)
    return jnp.transpose(out, (2, 0, 1))


# speedup vs baseline: 1.0277x; 1.0277x over previous
import functools
import math

import jax
import jax.numpy as jnp
from jax import lax
from jax.experimental import pallas as pl
from jax.experimental.pallas import tpu as pltpu
from jax.experimental.pallas import tpu_sc as plsc

VOCAB = 1000000
DIM = 32
SEQ = 200
NB = 4096
HALF = 16

NC = 2
NS = 16
DPC = DIM // NC
BS = 128
SLICES = NB // (NS * BS)
BLK = 40
NBLK = SEQ // BLK
ROW_CHUNK = 62464
ROW_LAST = 999936 - (NS - 1) * ROW_CHUNK
TAIL = 128

_SCALE = math.sqrt(float(DIM))


def _pe_table(length, depth):
    half = depth // 2
    positions = jnp.arange(length, dtype=jnp.float32).reshape(-1, 1)
    depths = jnp.arange(half, dtype=jnp.float32).reshape(1, -1) / half
    angle = positions / (10000.0 ** depths)
    return jnp.concatenate([jnp.sin(angle), jnp.cos(angle)], axis=-1)


@functools.lru_cache(maxsize=None)
def _make_kernel():
    mesh = plsc.VectorSubcoreMesh(core_axis_name="c", subcore_axis_name="s")

    @functools.partial(
        pl.kernel,
        mesh=mesh,
        out_type=jax.ShapeDtypeStruct((SEQ, DIM, NB), jnp.float32),
        scratch_types=[
            pltpu.VMEM((SLICES, SEQ, BS), jnp.int32),
            pltpu.VMEM((BLK, BS), jnp.float32),
            pltpu.VMEM((BLK, BS), jnp.float32),
            pltpu.VMEM((SEQ // 8 + 1, 128), jnp.float32),
            pltpu.VMEM((TAIL,), jnp.float32),
            pltpu.VMEM((TAIL,), jnp.int32),
            pltpu.VMEM_SHARED((VOCAB,), jnp.float32),
            pltpu.SemaphoreType.DMA,
            pltpu.SemaphoreType.DMA,
            pltpu.SemaphoreType.DMA,
        ],
    )
    def emb(xt_hbm, tt_hbm, pe_hbm, out_hbm, idx_v, val_a, val_b,
            pe_v, tail_v, tidx_v, row_sh, gsem, osem_a, osem_b):
        cid = lax.axis_index("c")
        sid = lax.axis_index("s")
        scale = jnp.float32(_SCALE)
        vals = (val_a, val_b)
        osems = (osem_a, osem_b)

        for jj in range(TAIL // HALF):
            tidx_v[pl.ds(jj * HALF, HALF)] = (
                lax.iota(jnp.int32, HALF) + (VOCAB - TAIL + jj * HALF)
            )

        for h in range(SLICES):
            b0 = (sid * SLICES + h) * BS
            pltpu.sync_copy(xt_hbm.at[:, pl.ds(b0, BS)], idx_v.at[h])

        def unit(dd, carry):
            d = cid * DPC + dd

            plsc.subcore_barrier()
            off = sid * ROW_CHUNK

            @pl.when(sid < NS - 1)
            def _chunk():
                pltpu.sync_copy(
                    tt_hbm.at[d, pl.ds(off, ROW_CHUNK)],
                    row_sh.at[pl.ds(off, ROW_CHUNK)],
                )

            @pl.when(sid == NS - 1)
            def _last():
                pltpu.sync_copy(
                    tt_hbm.at[d, pl.ds((NS - 1) * ROW_CHUNK, ROW_LAST)],
                    row_sh.at[pl.ds((NS - 1) * ROW_CHUNK, ROW_LAST)],
                )

            pltpu.sync_copy(pe_hbm.at[d], pe_v)

            @pl.when(sid == NS - 1)
            def _tail():
                for jj in range(TAIL // HALF):
                    tail_v[pl.ds(jj * HALF, HALF)] = (
                        pe_v[SEQ // 8, pl.ds(jj * HALF, HALF)]
                    )
                pltpu.sync_copy(tail_v, row_sh.at[tidx_v])

            plsc.subcore_barrier()

            def hj(bi):
                return bi // NBLK, bi % NBLK

            def out_blk(bi):
                h, j = hj(bi)
                b0 = (sid * SLICES + h) * BS
                return out_hbm.at[pl.ds(j * BLK, BLK), d, pl.ds(b0, BS)]

            gathers = {}

            def fire(bi):
                h, j = hj(bi)
                dst = vals[bi % 2]
                cps = []
                for k in range(BLK):
                    cps.append(pltpu.async_copy(
                        row_sh.at[idx_v.at[h, j * BLK + k]], dst.at[k], gsem
                    ))
                gathers[bi] = cps

            def drain_out(bi):
                cp = pltpu.make_async_copy(out_blk(bi), vals[bi % 2],
                                           osems[bi % 2])
                if bi < 2:
                    @pl.when(dd > 0)
                    def _w():
                        cp.wait()
                else:
                    cp.wait()

            def compute(bi):
                _, j = hj(bi)
                dst = vals[bi % 2]

                def body(sl, c2):
                    s_glob = j * BLK + sl
                    pe_vec = pe_v[s_glob // 8,
                                  pl.ds((s_glob % 8) * HALF, HALF)]
                    for jj in range(BS // HALF):
                        dst[sl, pl.ds(jj * HALF, HALF)] = (
                            dst[sl, pl.ds(jj * HALF, HALF)] * scale + pe_vec
                        )
                    return c2

                lax.fori_loop(0, BLK, body, 0, unroll=False)

            def store(bi):
                pltpu.async_copy(vals[bi % 2], out_blk(bi), osems[bi % 2])

            nbi = SLICES * NBLK
            drain_out(0)
            fire(0)
            for bi in range(nbi):
                for cp in gathers[bi]:
                    cp.wait()
                if bi < nbi - 1:
                    drain_out(bi + 1)
                    fire(bi + 1)
                compute(bi)
                store(bi)
            return carry

        lax.fori_loop(0, DPC, unit, 0, unroll=False)
        pltpu.make_async_copy(
            out_hbm.at[pl.ds(0, BLK), 0, pl.ds(0, BS)], val_a, osem_a
        ).wait()
        pltpu.make_async_copy(
            out_hbm.at[pl.ds(0, BLK), 0, pl.ds(0, BS)], val_b, osem_b
        ).wait()

    return emb


def kernel(x, table):
    batch, seq = x.shape
    vocab, dim = table.shape
    assert (batch, seq, vocab, dim) == (NB, SEQ, VOCAB, DIM)
    pe = _pe_table(seq, dim)
    pe_b = jnp.broadcast_to(
        pe.T.reshape(dim, seq // 8, 8, 1), (dim, seq // 8, 8, HALF)
    ).reshape(dim, seq // 8, 8 * HALF)
    tail = table.T[:, VOCAB - TAIL:].reshape(dim, 1, TAIL)
    pe_b = jnp.concatenate([pe_b, tail], axis=1)
    out = _make_kernel()(x.T, table.T, pe_b)
    return jnp.transpose(out, (2, 0, 1))
